# baseline (speedup 1.0000x reference)
import jax
import jax.numpy as jnp
from jax import lax
from jax.experimental import pallas as pl
from jax.experimental.pallas import tpu as pltpu
from jax.experimental.pallas import tpu_sc as plsc

D = 64
SCALE = 8.0
NC, NS, L = 2, 16, 16
NW = NC * NS
BW = 128
NBUF = 4


def _emb_body(S, idx_hbm, table_hbm, out_hbm, idx_v, rowbufs, tbufs, gsems, wsems):
    wid = lax.axis_index("s") * NC + lax.axis_index("c")
    b0 = wid * BW
    pltpu.sync_copy(idx_hbm.at[:, pl.ds(b0, BW)], idx_v)

    iota = lax.iota(jnp.int32, L)
    dvecs = [jnp.full((L,), 16 * k, jnp.int32) + iota for k in range(4)]

    def fire(s, slot):
        pltpu.async_copy(table_hbm.at[idx_v.at[s]], rowbufs[slot], gsems[slot])

    for slot in range(NBUF):
        fire(slot, slot)

    def round_body(g, carry):
        for u in range(NBUF):
            s = g * NBUF + u
            slot = u
            tsl = u % 2
            pltpu.make_async_copy(
                table_hbm.at[pl.ds(0, BW)], rowbufs[slot], gsems[slot]
            ).wait()

            if u >= 2:
                for i in range(D // 8):
                    pltpu.make_async_copy(
                        tbufs[tsl].at[pl.ds(8 * i, 8), pl.ds(0, BW)],
                        out_hbm.at[s - 2, i, wid],
                        wsems[tsl],
                    ).wait()
            else:

                @pl.when(g > 0)
                def _():
                    for i in range(D // 8):
                        pltpu.make_async_copy(
                            tbufs[tsl].at[pl.ds(8 * i, 8), pl.ds(0, BW)],
                            out_hbm.at[s - 2, i, wid],
                            wsems[tsl],
                        ).wait()

            @plsc.parallel_loop(0, BW, unroll=16)
            def transpose_scale(b):
                colv = jnp.full((L,), b, jnp.int32)
                for k in range(4):
                    v = rowbufs[slot][b, pl.ds(16 * k, L)] * SCALE
                    plsc.store_scatter(tbufs[tsl], [dvecs[k], colv], v)

            for i in range(D // 8):
                pltpu.async_copy(
                    tbufs[tsl].at[pl.ds(8 * i, 8), pl.ds(0, BW)],
                    out_hbm.at[s, i, wid],
                    wsems[tsl],
                )

            @pl.when(s + NBUF < S)
            def _():
                fire(s + NBUF, slot)

        return carry

    lax.fori_loop(0, S // NBUF, round_body, 0)

    for tsl in range(2):
        for i in range(D // 8):
            pltpu.make_async_copy(
                tbufs[tsl].at[pl.ds(8 * i, 8), pl.ds(0, BW)],
                out_hbm.at[S - 2 + tsl, i, wid],
                wsems[tsl],
            ).wait()


def kernel(tokens, table):
    B, S = tokens.shape
    assert B == NW * BW and S % NBUF == 0
    idx = tokens.T.astype(jnp.int32)
    mesh = plsc.VectorSubcoreMesh(core_axis_name="c", subcore_axis_name="s")
    out5 = pl.kernel(
        lambda *refs: _emb_body(S, *refs),
        out_type=jax.ShapeDtypeStruct((S, D // 8, B // 128, 8, 128), jnp.float32),
        mesh=mesh,
        compiler_params=pltpu.CompilerParams(
            use_tc_tiling_on_sc=False, needs_layout_passes=False
        ),
        scratch_types=[
            pltpu.VMEM((S, BW), jnp.int32),
            [pltpu.VMEM((BW, D), jnp.float32) for _ in range(NBUF)],
            [pltpu.VMEM((D, 129), jnp.float32) for _ in range(2)],
            [pltpu.SemaphoreType.DMA for _ in range(NBUF)],
            [pltpu.SemaphoreType.DMA for _ in range(2)],
        ],
    )(idx, table)
    return out5.transpose(2, 4, 0, 1, 3).reshape(B, S, D)


# speedup vs baseline: 1.0509x; 1.0509x over previous
import jax
import jax.numpy as jnp
from jax import lax
from jax.experimental import pallas as pl
from jax.experimental.pallas import tpu as pltpu
from jax.experimental.pallas import tpu_sc as plsc

D = 64
SCALE = 8.0
NC, NS, L = 2, 16, 16
NW = NC * NS
BW = 128
NBUF = 4


def _emb_body(S, idx_hbm, table_hbm, out_hbm, idx_v, rowbufs, tbufs, gsems, wsems):
    wid = lax.axis_index("s") * NC + lax.axis_index("c")
    b0 = wid * BW
    pltpu.sync_copy(idx_hbm.at[:, pl.ds(b0, BW)], idx_v)

    iota = lax.iota(jnp.int32, L)
    dvecs = [jnp.full((L,), 16 * k, jnp.int32) + iota for k in range(4)]

    def fire(s, slot):
        pltpu.async_copy(table_hbm.at[idx_v.at[s]], rowbufs[slot], gsems[slot])

    for slot in range(NBUF):
        fire(slot, slot)

    def round_body(g, carry):
        for u in range(NBUF):
            s = g * NBUF + u
            slot = u
            tsl = u % 2
            pltpu.make_async_copy(
                table_hbm.at[pl.ds(0, BW)], rowbufs[slot], gsems[slot]
            ).wait()

            if u >= 2:
                for i in range(D // 8):
                    pltpu.make_async_copy(
                        tbufs[tsl].at[pl.ds(8 * i, 8), pl.ds(0, BW)],
                        out_hbm.at[s - 2, i, wid],
                        wsems[tsl],
                    ).wait()
            else:

                @pl.when(g > 0)
                def _():
                    for i in range(D // 8):
                        pltpu.make_async_copy(
                            tbufs[tsl].at[pl.ds(8 * i, 8), pl.ds(0, BW)],
                            out_hbm.at[s - 2, i, wid],
                            wsems[tsl],
                        ).wait()

            @plsc.parallel_loop(0, BW, unroll=8)
            def transpose_scale(b):
                colv = jnp.full((L,), b, jnp.int32)
                for k in range(4):
                    v = rowbufs[slot][b, pl.ds(16 * k, L)] * SCALE
                    plsc.store_scatter(tbufs[tsl], [dvecs[k], colv], v)

            for i in range(D // 8):
                pltpu.async_copy(
                    tbufs[tsl].at[pl.ds(8 * i, 8), pl.ds(0, BW)],
                    out_hbm.at[s, i, wid],
                    wsems[tsl],
                )

            @pl.when(s + NBUF < S)
            def _():
                fire(s + NBUF, slot)

        return carry

    lax.fori_loop(0, S // NBUF, round_body, 0)

    for tsl in range(2):
        for i in range(D // 8):
            pltpu.make_async_copy(
                tbufs[tsl].at[pl.ds(8 * i, 8), pl.ds(0, BW)],
                out_hbm.at[S - 2 + tsl, i, wid],
                wsems[tsl],
            ).wait()


def kernel(tokens, table):
    B, S = tokens.shape
    assert B == NW * BW and S % NBUF == 0
    idx = tokens.T.astype(jnp.int32)
    mesh = plsc.VectorSubcoreMesh(core_axis_name="c", subcore_axis_name="s")
    out5 = pl.kernel(
        lambda *refs: _emb_body(S, *refs),
        out_type=jax.ShapeDtypeStruct((S, D // 8, B // 128, 8, 128), jnp.float32),
        mesh=mesh,
        compiler_params=pltpu.CompilerParams(
            use_tc_tiling_on_sc=False, needs_layout_passes=False
        ),
        scratch_types=[
            pltpu.VMEM((S, BW), jnp.int32),
            [pltpu.VMEM((BW, D), jnp.float32) for _ in range(NBUF)],
            [pltpu.VMEM((D, 129), jnp.float32) for _ in range(2)],
            [pltpu.SemaphoreType.DMA for _ in range(NBUF)],
            [pltpu.SemaphoreType.DMA for _ in range(2)],
        ],
    )(idx, table)
    return out5.transpose(2, 4, 0, 1, 3).reshape(B, S, D)
